# baseline (speedup 1.0000x reference)
import jax
import jax.numpy as jnp
from jax import lax
from jax.experimental import pallas as pl
from jax.experimental.pallas import tpu as pltpu
from jax.experimental.pallas import tpu_sc as plsc

B = 16384
D = 64
K = 20
VHALF = 500_000
NC = 2
NS = 16
NW = NC * NS
RPW = B // NW
NPW = RPW * K
CH = 16
NEGC = CH * K
NCHUNK = RPW // CH


def _sload(ref, i):
    return ref[pl.ds(i, 16)][0]


def _dot16(a_ref, a_row, a_off, b_ref, b_row, b_off):
    acc = a_ref[a_row, pl.ds(a_off, 16)] * b_ref[b_row, pl.ds(b_off, 16)]
    for c in range(1, D // 16):
        acc = acc + (a_ref[a_row, pl.ds(a_off + c * 16, 16)]
                     * b_ref[b_row, pl.ds(b_off + c * 16, 16)])
    return jnp.sum(acc)


def _split_idx(src_ref, n, row_ref, off_ref):
    def body(i, _):
        v = src_ref[pl.ds(i * 16, 16)]
        row_ref[pl.ds(i * 16, 16)] = lax.shift_right_logical(v, 1)
        off_ref[pl.ds(i * 16, 16)] = (v & 1) * 64
        return 0
    lax.fori_loop(0, n // 16, body, 0)


def _sc_body(centers_hbm, positives_hbm, negs_hbm, win_hbm, wout_hbm,
             pos_hbm, negsc_hbm,
             cidx, pidx, nidx, crow, coff, prow, poff, nrow, noff,
             cbuf, pbuf, nbuf, pos_o, neg_o, sem):
    wid = lax.axis_index("s") * NC + lax.axis_index("c")
    base = wid * RPW
    nbase = wid * NPW

    pltpu.sync_copy(centers_hbm.at[pl.ds(base, RPW)], cidx)
    pltpu.sync_copy(positives_hbm.at[pl.ds(base, RPW)], pidx)
    pltpu.sync_copy(negs_hbm.at[pl.ds(nbase, NPW)], nidx)
    _split_idx(cidx, RPW, crow, coff)
    _split_idx(pidx, RPW, prow, poff)
    _split_idx(nidx, NPW, nrow, noff)

    def chunk(j, _):
        ro = j * CH
        no = j * NEGC
        cps = [
            pltpu.async_copy(win_hbm.at[crow.at[pl.ds(ro, CH)]], cbuf, sem),
            pltpu.async_copy(wout_hbm.at[prow.at[pl.ds(ro, CH)]], pbuf, sem),
        ] + [
            pltpu.async_copy(wout_hbm.at[nrow.at[pl.ds(no + o, sz)]],
                             nbuf.at[pl.ds(o, sz)], sem)
            for o, sz in ((0, 128), (128, 128), (256, 64))
        ]
        for cp in cps:
            cp.wait()

        lanes = lax.iota(jnp.int32, 16)

        def row_body(r, pos_vec):
            row = ro + r
            hc = _sload(coff, row)
            pos_vec = jnp.where(lanes == r,
                                _dot16(cbuf, r, hc, pbuf, r, _sload(poff, row)),
                                pos_vec)
            v0 = jnp.zeros((16,), jnp.float32)
            v1 = jnp.zeros((16,), jnp.float32)
            for k in range(16):
                nr = r * K + k
                v0 = jnp.where(lanes == k,
                               _dot16(cbuf, r, hc, nbuf, nr, _sload(noff, no + nr)), v0)
            for k in range(16, K):
                nr = r * K + k
                v1 = jnp.where(lanes == k - 16,
                               _dot16(cbuf, r, hc, nbuf, nr, _sload(noff, no + nr)), v1)
            nb = no + r * K
            plsc.store_scatter(neg_o, [nb + lanes], v0)
            plsc.store_scatter(neg_o, [nb + 16 + lanes], v1, mask=lanes < K - 16)
            return pos_vec

        pos_vec = lax.fori_loop(0, CH, row_body, jnp.zeros((16,), jnp.float32))
        pos_o[pl.ds(ro, CH)] = pos_vec
        return 0

    lax.fori_loop(0, NCHUNK, chunk, 0)

    pltpu.sync_copy(pos_o, pos_hbm.at[pl.ds(base, RPW)])
    pltpu.sync_copy(neg_o, negsc_hbm.at[pl.ds(nbase, NPW)])


_sc_scores = pl.kernel(
    _sc_body,
    out_type=(
        jax.ShapeDtypeStruct((B,), jnp.float32),
        jax.ShapeDtypeStruct((B * K,), jnp.float32),
    ),
    mesh=plsc.VectorSubcoreMesh(
        core_axis_name="c", subcore_axis_name="s",
        num_cores=NC, num_subcores=NS,
    ),
    compiler_params=pltpu.CompilerParams(needs_layout_passes=False),
    scratch_types=[
        pltpu.VMEM((RPW,), jnp.int32),
        pltpu.VMEM((RPW,), jnp.int32),
        pltpu.VMEM((NPW,), jnp.int32),
        pltpu.VMEM((RPW,), jnp.int32),
        pltpu.VMEM((RPW + 16,), jnp.int32),
        pltpu.VMEM((RPW,), jnp.int32),
        pltpu.VMEM((RPW + 16,), jnp.int32),
        pltpu.VMEM((NPW,), jnp.int32),
        pltpu.VMEM((NPW + 16,), jnp.int32),
        pltpu.VMEM((CH, 2 * D), jnp.float32),
        pltpu.VMEM((CH, 2 * D), jnp.float32),
        pltpu.VMEM((NEGC, 2 * D), jnp.float32),
        pltpu.VMEM((RPW,), jnp.float32),
        pltpu.VMEM((NPW,), jnp.float32),
        pltpu.SemaphoreType.DMA,
    ],
)


def _log_sigmoid(x):
    return jnp.minimum(x, 0.0) - jnp.log1p(jnp.exp(-jnp.abs(x)))


def _loss_body(pos_ref, neg_ref, out_ref):
    ls_pos = jnp.sum(_log_sigmoid(pos_ref[...]))
    ls_neg = jnp.sum(_log_sigmoid(-neg_ref[...]))
    out_ref[0, 0] = -(ls_pos + ls_neg) / B


_loss = pl.pallas_call(
    _loss_body,
    out_shape=jax.ShapeDtypeStruct((1, 1), jnp.float32),
    out_specs=pl.BlockSpec(memory_space=pltpu.SMEM),
)


@jax.jit
def kernel(centers, positives, negatives, W_in, W_out):
    pos_s, neg_s = _sc_scores(centers, positives, negatives.reshape(-1),
                              W_in.reshape(VHALF, 2 * D),
                              W_out.reshape(VHALF, 2 * D))
    total = _loss(pos_s.reshape(128, B // 128), neg_s.reshape(B * K // 128, 128))
    return total[0, 0]


# speedup vs baseline: 1.8043x; 1.8043x over previous
import jax
import jax.numpy as jnp
from jax import lax
from jax.experimental import pallas as pl
from jax.experimental.pallas import tpu as pltpu
from jax.experimental.pallas import tpu_sc as plsc

B = 16384
D = 64
K = 20
V = 1_000_000
CONVR = 2048
NCONV = (V + 2 * CONVR - 1) // (2 * CONVR)
SROWS = NCONV * CONVR
CLAMPB = (V - 1) // CONVR
NC = 2
NS = 16
NW = NC * NS
RPW = B // NW
NPW = RPW * K
CH = 16
NEGC = CH * K
NCHUNK = RPW // CH


def _sload(ref, i):
    return ref[pl.ds(i, 16)][0]


def _dot16(a_ref, a_row, a_off, b_ref, b_row, b_off):
    acc = a_ref[a_row, pl.ds(a_off, 16)] * b_ref[b_row, pl.ds(b_off, 16)]
    for c in range(1, D // 16):
        acc = acc + (a_ref[a_row, pl.ds(a_off + c * 16, 16)]
                     * b_ref[b_row, pl.ds(b_off + c * 16, 16)])
    return jnp.sum(acc)


def _split_idx(src_ref, n, row_ref, off_ref):
    def body(i, _):
        v = src_ref[pl.ds(i * 16, 16)]
        u = v & (2 * CONVR - 1)
        row_ref[pl.ds(i * 16, 16)] = (
            lax.shift_right_logical(v, 12) * CONVR + (u & (CONVR - 1)))
        off_ref[pl.ds(i * 16, 16)] = jnp.where(u >= CONVR, 64, 0)
        return 0
    lax.fori_loop(0, n // 16, body, 0)


def _sc_body(centers_hbm, positives_hbm, negs_hbm, win_hbm, wout_hbm,
             pos_hbm, negsc_hbm,
             cidx, pidx, nidx, crow, coff, prow, poff, nrow, noff,
             cbuf, pbuf, nbuf, pos_o, neg_o, sem):
    wid = lax.axis_index("s") * NC + lax.axis_index("c")
    base = wid * RPW
    nbase = wid * NPW

    pltpu.sync_copy(centers_hbm.at[pl.ds(base, RPW)], cidx)
    pltpu.sync_copy(positives_hbm.at[pl.ds(base, RPW)], pidx)
    pltpu.sync_copy(negs_hbm.at[pl.ds(nbase, NPW)], nidx)
    _split_idx(cidx, RPW, crow, coff)
    _split_idx(pidx, RPW, prow, poff)
    _split_idx(nidx, NPW, nrow, noff)

    def chunk(j, _):
        ro = j * CH
        no = j * NEGC
        cps = [
            pltpu.async_copy(win_hbm.at[crow.at[pl.ds(ro, CH)]], cbuf, sem),
            pltpu.async_copy(wout_hbm.at[prow.at[pl.ds(ro, CH)]], pbuf, sem),
        ] + [
            pltpu.async_copy(wout_hbm.at[nrow.at[pl.ds(no + o, sz)]],
                             nbuf.at[pl.ds(o, sz)], sem)
            for o, sz in ((0, 128), (128, 128), (256, 64))
        ]
        for cp in cps:
            cp.wait()

        lanes = lax.iota(jnp.int32, 16)

        def row_body(r, pos_vec):
            row = ro + r
            hc = _sload(coff, row)
            pos_vec = jnp.where(lanes == r,
                                _dot16(cbuf, r, hc, pbuf, r, _sload(poff, row)),
                                pos_vec)
            v0 = jnp.zeros((16,), jnp.float32)
            v1 = jnp.zeros((16,), jnp.float32)
            for k in range(16):
                nr = r * K + k
                v0 = jnp.where(lanes == k,
                               _dot16(cbuf, r, hc, nbuf, nr, _sload(noff, no + nr)), v0)
            for k in range(16, K):
                nr = r * K + k
                v1 = jnp.where(lanes == k - 16,
                               _dot16(cbuf, r, hc, nbuf, nr, _sload(noff, no + nr)), v1)
            nb = no + r * K
            plsc.store_scatter(neg_o, [nb + lanes], v0)
            plsc.store_scatter(neg_o, [nb + 16 + lanes], v1, mask=lanes < K - 16)
            return pos_vec

        pos_vec = lax.fori_loop(0, CH, row_body, jnp.zeros((16,), jnp.float32))
        pos_o[pl.ds(ro, CH)] = pos_vec
        return 0

    lax.fori_loop(0, NCHUNK, chunk, 0)

    pltpu.sync_copy(pos_o, pos_hbm.at[pl.ds(base, RPW)])
    pltpu.sync_copy(neg_o, negsc_hbm.at[pl.ds(nbase, NPW)])


_sc_scores = pl.kernel(
    _sc_body,
    out_type=(
        jax.ShapeDtypeStruct((B,), jnp.float32),
        jax.ShapeDtypeStruct((B * K,), jnp.float32),
    ),
    mesh=plsc.VectorSubcoreMesh(
        core_axis_name="c", subcore_axis_name="s",
        num_cores=NC, num_subcores=NS,
    ),
    compiler_params=pltpu.CompilerParams(needs_layout_passes=False),
    scratch_types=[
        pltpu.VMEM((RPW,), jnp.int32),
        pltpu.VMEM((RPW,), jnp.int32),
        pltpu.VMEM((NPW,), jnp.int32),
        pltpu.VMEM((RPW,), jnp.int32),
        pltpu.VMEM((RPW + 16,), jnp.int32),
        pltpu.VMEM((RPW,), jnp.int32),
        pltpu.VMEM((RPW + 16,), jnp.int32),
        pltpu.VMEM((NPW,), jnp.int32),
        pltpu.VMEM((NPW + 16,), jnp.int32),
        pltpu.VMEM((CH, 2 * D), jnp.float32),
        pltpu.VMEM((CH, 2 * D), jnp.float32),
        pltpu.VMEM((NEGC, 2 * D), jnp.float32),
        pltpu.VMEM((RPW,), jnp.float32),
        pltpu.VMEM((NPW,), jnp.float32),
        pltpu.SemaphoreType.DMA,
    ],
)


def _log_sigmoid(x):
    return jnp.minimum(x, 0.0) - jnp.log1p(jnp.exp(-jnp.abs(x)))


def _loss_body(pos_ref, neg_ref, out_ref):
    ls_pos = jnp.sum(_log_sigmoid(pos_ref[...]))
    ls_neg = jnp.sum(_log_sigmoid(-neg_ref[...]))
    out_ref[0, 0] = -(ls_pos + ls_neg) / B


_loss = pl.pallas_call(
    _loss_body,
    out_shape=jax.ShapeDtypeStruct((1, 1), jnp.float32),
    out_specs=pl.BlockSpec(memory_space=pltpu.SMEM),
)


def _conv_body(a1_ref, a2_ref, b1_ref, b2_ref, oa_ref, ob_ref):
    oa_ref[...] = jnp.concatenate([a1_ref[...].T, a2_ref[...].T], axis=1)
    ob_ref[...] = jnp.concatenate([b1_ref[...].T, b2_ref[...].T], axis=1)


_convert = pl.pallas_call(
    _conv_body,
    grid=(NCONV,),
    in_specs=[
        pl.BlockSpec((D, CONVR), lambda j: (0, 2 * j)),
        pl.BlockSpec((D, CONVR), lambda j: (0, jnp.minimum(2 * j + 1, CLAMPB))),
        pl.BlockSpec((D, CONVR), lambda j: (0, 2 * j)),
        pl.BlockSpec((D, CONVR), lambda j: (0, jnp.minimum(2 * j + 1, CLAMPB))),
    ],
    out_specs=[
        pl.BlockSpec((CONVR, 2 * D), lambda j: (j, 0)),
        pl.BlockSpec((CONVR, 2 * D), lambda j: (j, 0)),
    ],
    out_shape=[
        jax.ShapeDtypeStruct((SROWS, 2 * D), jnp.float32),
        jax.ShapeDtypeStruct((SROWS, 2 * D), jnp.float32),
    ],
)


@jax.jit
def kernel(centers, positives, negatives, W_in, W_out):
    win_t, wout_t = W_in.T, W_out.T
    s_in, s_out = _convert(win_t, win_t, wout_t, wout_t)
    pos_s, neg_s = _sc_scores(centers, positives, negatives.reshape(-1),
                              s_in, s_out)
    total = _loss(pos_s.reshape(128, B // 128), neg_s.reshape(B * K // 128, 128))
    return total[0, 0]


# speedup vs baseline: 2.0239x; 1.1217x over previous
import jax
import jax.numpy as jnp
from jax import lax
from jax.experimental import pallas as pl
from jax.experimental.pallas import tpu as pltpu
from jax.experimental.pallas import tpu_sc as plsc

B = 16384
D = 64
K = 20
V = 1_000_000
CONVR = 2048
NCONV = (V + 2 * CONVR - 1) // (2 * CONVR)
SROWS = NCONV * CONVR
CLAMPB = (V - 1) // CONVR
NC = 2
NS = 16
NW = NC * NS
RPW = B // NW
NPW = RPW * K
CH = 16
NEGC = CH * K
NCHUNK = RPW // CH


def _sload(ref, i):
    return ref[pl.ds(i, 16)][0]


def _dotv(cv, b_ref, b_row, b_off):
    acc = cv[0] * b_ref[b_row, pl.ds(b_off, 16)]
    for c in range(1, D // 16):
        acc = acc + cv[c] * b_ref[b_row, pl.ds(b_off + c * 16, 16)]
    return jnp.sum(acc)


def _split_idx(idx_ref, n, off_ref):
    def body(i, _):
        v = idx_ref[pl.ds(i * 16, 16)]
        u = v & (2 * CONVR - 1)
        idx_ref[pl.ds(i * 16, 16)] = (
            lax.shift_right_logical(v, 12) * CONVR + (u & (CONVR - 1)))
        off_ref[pl.ds(i * 16, 16)] = jnp.where(u >= CONVR, 64, 0)
        return 0
    lax.fori_loop(0, n // 16, body, 0)


def _sc_body(centers_hbm, positives_hbm, negs_hbm, win_hbm, wout_hbm,
             pos_hbm, negsc_hbm,
             cidx, coff, pidx, poff, nidx, noff,
             cb0, cb1, pb0, pb1, nb0, nb1, pos_o, neg_o, sem0, sem1):
    wid = lax.axis_index("s") * NC + lax.axis_index("c")
    base = wid * RPW
    nbase = wid * NPW

    pltpu.sync_copy(centers_hbm.at[pl.ds(base, RPW)], cidx)
    pltpu.sync_copy(positives_hbm.at[pl.ds(base, RPW)], pidx)
    pltpu.sync_copy(negs_hbm.at[pl.ds(nbase, NPW)], nidx)
    _split_idx(cidx, RPW, coff)
    _split_idx(pidx, RPW, poff)
    _split_idx(nidx, NPW, noff)

    bufs = ((cb0, pb0, nb0, sem0), (cb1, pb1, nb1, sem1))
    NSL = ((0, 128), (128, 128), (256, 64))

    def copies(j, slot):
        cb, pb, nb, sem = bufs[slot]
        ro = j * CH
        no = j * NEGC
        return (
            [(win_hbm.at[cidx.at[pl.ds(ro, CH)]], cb, sem),
             (wout_hbm.at[pidx.at[pl.ds(ro, CH)]], pb, sem)]
            + [(wout_hbm.at[nidx.at[pl.ds(no + o, sz)]],
                nb.at[pl.ds(o, sz)], sem) for o, sz in NSL]
        )

    def fire(j, slot):
        for src, dst, sem in copies(j, slot):
            pltpu.async_copy(src, dst, sem)

    def drain(j, slot):
        for src, dst, sem in copies(j, slot):
            pltpu.make_async_copy(src, dst, sem).wait()

    lanes = lax.iota(jnp.int32, 16)
    zero16 = jnp.zeros((16,), jnp.float32)

    def compute(j, slot):
        cb, pb, nb, _ = bufs[slot]
        ro = j * CH
        no = j * NEGC

        def row_body(r, pos_vec):
            hc = _sload(coff, ro + r)
            cv = tuple(cb[r, pl.ds(hc + c * 16, 16)] for c in range(D // 16))
            pos_vec = jnp.where(lanes == r,
                                _dotv(cv, pb, r, _sload(poff, ro + r)),
                                pos_vec)
            nb_off = no + r * K
            ov0 = noff[pl.ds(nb_off, 16)]
            ov1 = noff[pl.ds(nb_off + 16, 16)]
            v0 = zero16
            v1 = zero16
            for k in range(16):
                v0 = jnp.where(lanes == k,
                               _dotv(cv, nb, r * K + k, ov0[k]), v0)
            for k in range(K - 16):
                v1 = jnp.where(lanes == k,
                               _dotv(cv, nb, r * K + 16 + k, ov1[k]), v1)
            plsc.store_scatter(neg_o, [nb_off + lanes], v0)
            plsc.store_scatter(neg_o, [nb_off + 16 + lanes], v1,
                               mask=lanes < K - 16)
            return pos_vec

        pos_vec = lax.fori_loop(0, CH, row_body, zero16)
        pos_o[pl.ds(ro, CH)] = pos_vec

    fire(0, 0)

    def pair(j2, _):
        j0 = 2 * j2
        drain(j0, 0)
        fire(j0 + 1, 1)
        compute(j0, 0)
        j1 = j0 + 1
        drain(j1, 1)

        @pl.when(j1 < NCHUNK - 1)
        def _():
            fire(j1 + 1, 0)

        compute(j1, 1)
        return 0

    lax.fori_loop(0, NCHUNK // 2, pair, 0)

    pltpu.sync_copy(pos_o, pos_hbm.at[pl.ds(base, RPW)])
    pltpu.sync_copy(neg_o, negsc_hbm.at[pl.ds(nbase, NPW)])


_sc_scores = pl.kernel(
    _sc_body,
    out_type=(
        jax.ShapeDtypeStruct((B,), jnp.float32),
        jax.ShapeDtypeStruct((B * K,), jnp.float32),
    ),
    mesh=plsc.VectorSubcoreMesh(
        core_axis_name="c", subcore_axis_name="s",
        num_cores=NC, num_subcores=NS,
    ),
    compiler_params=pltpu.CompilerParams(needs_layout_passes=False),
    scratch_types=[
        pltpu.VMEM((RPW,), jnp.int32),
        pltpu.VMEM((RPW + 16,), jnp.int32),
        pltpu.VMEM((RPW,), jnp.int32),
        pltpu.VMEM((RPW + 16,), jnp.int32),
        pltpu.VMEM((NPW,), jnp.int32),
        pltpu.VMEM((NPW + 16,), jnp.int32),
        pltpu.VMEM((CH, 2 * D), jnp.float32),
        pltpu.VMEM((CH, 2 * D), jnp.float32),
        pltpu.VMEM((CH, 2 * D), jnp.float32),
        pltpu.VMEM((CH, 2 * D), jnp.float32),
        pltpu.VMEM((NEGC, 2 * D), jnp.float32),
        pltpu.VMEM((NEGC, 2 * D), jnp.float32),
        pltpu.VMEM((RPW,), jnp.float32),
        pltpu.VMEM((NPW,), jnp.float32),
        pltpu.SemaphoreType.DMA,
        pltpu.SemaphoreType.DMA,
    ],
)


def _log_sigmoid(x):
    return jnp.minimum(x, 0.0) - jnp.log1p(jnp.exp(-jnp.abs(x)))


def _loss_body(pos_ref, neg_ref, out_ref):
    ls_pos = jnp.sum(_log_sigmoid(pos_ref[...]))
    ls_neg = jnp.sum(_log_sigmoid(-neg_ref[...]))
    out_ref[0, 0] = -(ls_pos + ls_neg) / B


_loss = pl.pallas_call(
    _loss_body,
    out_shape=jax.ShapeDtypeStruct((1, 1), jnp.float32),
    out_specs=pl.BlockSpec(memory_space=pltpu.SMEM),
)


def _conv_body(a1_ref, a2_ref, b1_ref, b2_ref, oa_ref, ob_ref):
    ident = (lax.broadcasted_iota(jnp.int32, (D, D), 0)
             == lax.broadcasted_iota(jnp.int32, (D, D), 1)).astype(jnp.float32)

    def t(x):
        return lax.dot_general(x, ident, (((0,), (0,)), ((), ())),
                               preferred_element_type=jnp.float32)

    oa_ref[...] = jnp.concatenate([t(a1_ref[...]), t(a2_ref[...])], axis=1)
    ob_ref[...] = jnp.concatenate([t(b1_ref[...]), t(b2_ref[...])], axis=1)


_convert = pl.pallas_call(
    _conv_body,
    grid=(NCONV,),
    in_specs=[
        pl.BlockSpec((D, CONVR), lambda j: (0, 2 * j)),
        pl.BlockSpec((D, CONVR), lambda j: (0, jnp.minimum(2 * j + 1, CLAMPB))),
        pl.BlockSpec((D, CONVR), lambda j: (0, 2 * j)),
        pl.BlockSpec((D, CONVR), lambda j: (0, jnp.minimum(2 * j + 1, CLAMPB))),
    ],
    out_specs=[
        pl.BlockSpec((CONVR, 2 * D), lambda j: (j, 0)),
        pl.BlockSpec((CONVR, 2 * D), lambda j: (j, 0)),
    ],
    out_shape=[
        jax.ShapeDtypeStruct((SROWS, 2 * D), jnp.float32),
        jax.ShapeDtypeStruct((SROWS, 2 * D), jnp.float32),
    ],
    compiler_params=pltpu.CompilerParams(fuse_transposed_lhs_in_matmul=True),
)


@jax.jit
def kernel(centers, positives, negatives, W_in, W_out):
    win_t, wout_t = W_in.T, W_out.T
    s_in, s_out = _convert(win_t, win_t, wout_t, wout_t)
    pos_s, neg_s = _sc_scores(centers, positives, negatives.reshape(-1),
                              s_in, s_out)
    total = _loss(pos_s.reshape(128, B // 128), neg_s.reshape(B * K // 128, 128))
    return total[0, 0]


# speedup vs baseline: 2.2982x; 1.1355x over previous
import jax
import jax.numpy as jnp
from jax import lax
from jax.experimental import pallas as pl
from jax.experimental.pallas import tpu as pltpu
from jax.experimental.pallas import tpu_sc as plsc

B = 16384
D = 64
K = 20
V = 1_000_000
CONVR = 4096
NCONV = (V + 2 * CONVR - 1) // (2 * CONVR)
SROWS = NCONV * CONVR
CLAMPB = (V - 1) // CONVR
_CONV_SHIFT = (2 * CONVR).bit_length() - 1
NC = 2
NS = 16
NW = NC * NS
RPW = B // NW
NPW = RPW * K
CH = 16
NEGC = CH * K
NCHUNK = RPW // CH


def _sload(ref, i):
    return ref[pl.ds(i, 16)][0]


def _dotv(cv, b_ref, b_row, b_off):
    acc = cv[0] * b_ref[b_row, pl.ds(b_off, 16)]
    for c in range(1, D // 16):
        acc = acc + cv[c] * b_ref[b_row, pl.ds(b_off + c * 16, 16)]
    return jnp.sum(acc)


def _split_idx(idx_ref, n, off_ref):
    def body(i, _):
        v = idx_ref[pl.ds(i * 16, 16)]
        u = v & (2 * CONVR - 1)
        idx_ref[pl.ds(i * 16, 16)] = (
            lax.shift_right_logical(v, _CONV_SHIFT) * CONVR + (u & (CONVR - 1)))
        off_ref[pl.ds(i * 16, 16)] = jnp.where(u >= CONVR, 64, 0)
        return 0
    lax.fori_loop(0, n // 16, body, 0)


def _sc_body(centers_hbm, positives_hbm, negs_hbm, win_hbm, wout_hbm,
             pos_hbm, negsc_hbm,
             cidx, coff, pidx, poff, nidx, noff,
             cb0, cb1, pb0, pb1, nb0, nb1, pos_o, neg_o, sem0, sem1):
    wid = lax.axis_index("s") * NC + lax.axis_index("c")
    base = wid * RPW
    nbase = wid * NPW

    pltpu.sync_copy(centers_hbm.at[pl.ds(base, RPW)], cidx)
    pltpu.sync_copy(positives_hbm.at[pl.ds(base, RPW)], pidx)
    pltpu.sync_copy(negs_hbm.at[pl.ds(nbase, NPW)], nidx)
    _split_idx(cidx, RPW, coff)
    _split_idx(pidx, RPW, poff)
    _split_idx(nidx, NPW, noff)

    bufs = ((cb0, pb0, nb0, sem0), (cb1, pb1, nb1, sem1))
    NSL = ((0, 128), (128, 128), (256, 64))

    def copies(j, slot):
        cb, pb, nb, sem = bufs[slot]
        ro = j * CH
        no = j * NEGC
        return (
            [(win_hbm.at[cidx.at[pl.ds(ro, CH)]], cb, sem),
             (wout_hbm.at[pidx.at[pl.ds(ro, CH)]], pb, sem)]
            + [(wout_hbm.at[nidx.at[pl.ds(no + o, sz)]],
                nb.at[pl.ds(o, sz)], sem) for o, sz in NSL]
        )

    def fire(j, slot):
        for src, dst, sem in copies(j, slot):
            pltpu.async_copy(src, dst, sem)

    def drain(j, slot):
        for src, dst, sem in copies(j, slot):
            pltpu.make_async_copy(src, dst, sem).wait()

    lanes = lax.iota(jnp.int32, 16)
    zero16 = jnp.zeros((16,), jnp.float32)

    def compute(j, slot):
        cb, pb, nb, _ = bufs[slot]
        ro = j * CH
        no = j * NEGC

        def row_body(r, pos_vec):
            hc = _sload(coff, ro + r)
            cv = tuple(cb[r, pl.ds(hc + c * 16, 16)] for c in range(D // 16))
            pos_vec = jnp.where(lanes == r,
                                _dotv(cv, pb, r, _sload(poff, ro + r)),
                                pos_vec)
            nb_off = no + r * K
            ov0 = noff[pl.ds(nb_off, 16)]
            ov1 = noff[pl.ds(nb_off + 16, 16)]
            v0 = zero16
            v1 = zero16
            for k in range(16):
                v0 = jnp.where(lanes == k,
                               _dotv(cv, nb, r * K + k, ov0[k]), v0)
            for k in range(K - 16):
                v1 = jnp.where(lanes == k,
                               _dotv(cv, nb, r * K + 16 + k, ov1[k]), v1)
            plsc.store_scatter(neg_o, [nb_off + lanes], v0)
            plsc.store_scatter(neg_o, [nb_off + 16 + lanes], v1,
                               mask=lanes < K - 16)
            return pos_vec

        pos_vec = lax.fori_loop(0, CH, row_body, zero16)
        pos_o[pl.ds(ro, CH)] = pos_vec

    fire(0, 0)

    def pair(j2, _):
        j0 = 2 * j2
        drain(j0, 0)
        fire(j0 + 1, 1)
        compute(j0, 0)
        j1 = j0 + 1
        drain(j1, 1)

        @pl.when(j1 < NCHUNK - 1)
        def _():
            fire(j1 + 1, 0)

        compute(j1, 1)
        return 0

    lax.fori_loop(0, NCHUNK // 2, pair, 0)

    pltpu.sync_copy(pos_o, pos_hbm.at[pl.ds(base, RPW)])
    pltpu.sync_copy(neg_o, negsc_hbm.at[pl.ds(nbase, NPW)])


_sc_scores = pl.kernel(
    _sc_body,
    out_type=(
        jax.ShapeDtypeStruct((B,), jnp.float32),
        jax.ShapeDtypeStruct((B * K,), jnp.float32),
    ),
    mesh=plsc.VectorSubcoreMesh(
        core_axis_name="c", subcore_axis_name="s",
        num_cores=NC, num_subcores=NS,
    ),
    compiler_params=pltpu.CompilerParams(needs_layout_passes=False),
    scratch_types=[
        pltpu.VMEM((RPW,), jnp.int32),
        pltpu.VMEM((RPW + 16,), jnp.int32),
        pltpu.VMEM((RPW,), jnp.int32),
        pltpu.VMEM((RPW + 16,), jnp.int32),
        pltpu.VMEM((NPW,), jnp.int32),
        pltpu.VMEM((NPW + 16,), jnp.int32),
        pltpu.VMEM((CH, 2 * D), jnp.float32),
        pltpu.VMEM((CH, 2 * D), jnp.float32),
        pltpu.VMEM((CH, 2 * D), jnp.float32),
        pltpu.VMEM((CH, 2 * D), jnp.float32),
        pltpu.VMEM((NEGC, 2 * D), jnp.float32),
        pltpu.VMEM((NEGC, 2 * D), jnp.float32),
        pltpu.VMEM((RPW,), jnp.float32),
        pltpu.VMEM((NPW,), jnp.float32),
        pltpu.SemaphoreType.DMA,
        pltpu.SemaphoreType.DMA,
    ],
)


def _log_sigmoid(x):
    return jnp.minimum(x, 0.0) - jnp.log1p(jnp.exp(-jnp.abs(x)))


def _loss_body(pos_ref, neg_ref, out_ref):
    ls_pos = jnp.sum(_log_sigmoid(pos_ref[...]))
    ls_neg = jnp.sum(_log_sigmoid(-neg_ref[...]))
    out_ref[0, 0] = -(ls_pos + ls_neg) / B


_loss = pl.pallas_call(
    _loss_body,
    out_shape=jax.ShapeDtypeStruct((1, 1), jnp.float32),
    out_specs=pl.BlockSpec(memory_space=pltpu.SMEM),
)


def _conv_body(a1_ref, a2_ref, b1_ref, b2_ref, oa_ref, ob_ref):
    ident = (lax.broadcasted_iota(jnp.int32, (D, D), 0)
             == lax.broadcasted_iota(jnp.int32, (D, D), 1)).astype(jnp.float32)

    def t(x):
        return lax.dot_general(x, ident, (((0,), (0,)), ((), ())),
                               preferred_element_type=jnp.float32)

    oa_ref[...] = jnp.concatenate([t(a1_ref[...]), t(a2_ref[...])], axis=1)
    ob_ref[...] = jnp.concatenate([t(b1_ref[...]), t(b2_ref[...])], axis=1)


_convert = pl.pallas_call(
    _conv_body,
    grid=(NCONV,),
    in_specs=[
        pl.BlockSpec((D, CONVR), lambda j: (0, 2 * j)),
        pl.BlockSpec((D, CONVR), lambda j: (0, jnp.minimum(2 * j + 1, CLAMPB))),
        pl.BlockSpec((D, CONVR), lambda j: (0, 2 * j)),
        pl.BlockSpec((D, CONVR), lambda j: (0, jnp.minimum(2 * j + 1, CLAMPB))),
    ],
    out_specs=[
        pl.BlockSpec((CONVR, 2 * D), lambda j: (j, 0)),
        pl.BlockSpec((CONVR, 2 * D), lambda j: (j, 0)),
    ],
    out_shape=[
        jax.ShapeDtypeStruct((SROWS, 2 * D), jnp.float32),
        jax.ShapeDtypeStruct((SROWS, 2 * D), jnp.float32),
    ],
    compiler_params=pltpu.CompilerParams(fuse_transposed_lhs_in_matmul=True),
)


@jax.jit
def kernel(centers, positives, negatives, W_in, W_out):
    win_t, wout_t = W_in.T, W_out.T
    s_in, s_out = _convert(win_t, win_t, wout_t, wout_t)
    pos_s, neg_s = _sc_scores(centers, positives, negatives.reshape(-1),
                              s_in, s_out)
    total = _loss(pos_s.reshape(128, B // 128), neg_s.reshape(B * K // 128, 128))
    return total[0, 0]


# speedup vs baseline: 2.3297x; 1.0137x over previous
import jax
import jax.numpy as jnp
from jax import lax
from jax.experimental import pallas as pl
from jax.experimental.pallas import tpu as pltpu
from jax.experimental.pallas import tpu_sc as plsc

B = 16384
D = 64
K = 20
V = 1_000_000
CONVR = 8192
NCONV = (V + 2 * CONVR - 1) // (2 * CONVR)
SROWS = NCONV * CONVR
CLAMPB = (V - 1) // CONVR
_CONV_SHIFT = (2 * CONVR).bit_length() - 1
NC = 2
NS = 16
NW = NC * NS
RPW = B // NW
NPW = RPW * K
CH = 16
NEGC = CH * K
NCHUNK = RPW // CH


def _sload(ref, i):
    return ref[pl.ds(i, 16)][0]


def _dotv(cv, b_ref, b_row, b_off):
    acc = cv[0] * b_ref[b_row, pl.ds(b_off, 16)]
    for c in range(1, D // 16):
        acc = acc + cv[c] * b_ref[b_row, pl.ds(b_off + c * 16, 16)]
    return jnp.sum(acc)


def _split_idx(idx_ref, n, off_ref):
    def body(i, _):
        v = idx_ref[pl.ds(i * 16, 16)]
        u = v & (2 * CONVR - 1)
        idx_ref[pl.ds(i * 16, 16)] = (
            lax.shift_right_logical(v, _CONV_SHIFT) * CONVR + (u & (CONVR - 1)))
        off_ref[pl.ds(i * 16, 16)] = jnp.where(u >= CONVR, 64, 0)
        return 0
    lax.fori_loop(0, n // 16, body, 0)


def _sc_body(centers_hbm, positives_hbm, negs_hbm, win_hbm, wout_hbm,
             pos_hbm, negsc_hbm,
             cidx, coff, pidx, poff, nidx, noff,
             cb0, cb1, pb0, pb1, nb0, nb1, pos_o, neg_o, sem0, sem1):
    wid = lax.axis_index("s") * NC + lax.axis_index("c")
    base = wid * RPW
    nbase = wid * NPW

    pltpu.sync_copy(centers_hbm.at[pl.ds(base, RPW)], cidx)
    pltpu.sync_copy(positives_hbm.at[pl.ds(base, RPW)], pidx)
    pltpu.sync_copy(negs_hbm.at[pl.ds(nbase, NPW)], nidx)
    _split_idx(cidx, RPW, coff)
    _split_idx(pidx, RPW, poff)
    _split_idx(nidx, NPW, noff)

    bufs = ((cb0, pb0, nb0, sem0), (cb1, pb1, nb1, sem1))
    NSL = ((0, 128), (128, 128), (256, 64))

    def copies(j, slot):
        cb, pb, nb, sem = bufs[slot]
        ro = j * CH
        no = j * NEGC
        return (
            [(win_hbm.at[cidx.at[pl.ds(ro, CH)]], cb, sem),
             (wout_hbm.at[pidx.at[pl.ds(ro, CH)]], pb, sem)]
            + [(wout_hbm.at[nidx.at[pl.ds(no + o, sz)]],
                nb.at[pl.ds(o, sz)], sem) for o, sz in NSL]
        )

    def fire(j, slot):
        for src, dst, sem in copies(j, slot):
            pltpu.async_copy(src, dst, sem)

    def drain(j, slot):
        for src, dst, sem in copies(j, slot):
            pltpu.make_async_copy(src, dst, sem).wait()

    lanes = lax.iota(jnp.int32, 16)
    zero16 = jnp.zeros((16,), jnp.float32)

    def compute(j, slot):
        cb, pb, nb, _ = bufs[slot]
        ro = j * CH
        no = j * NEGC

        def row_body(r, pos_vec):
            hc = _sload(coff, ro + r)
            cv = tuple(cb[r, pl.ds(hc + c * 16, 16)] for c in range(D // 16))
            pos_vec = jnp.where(lanes == r,
                                _dotv(cv, pb, r, _sload(poff, ro + r)),
                                pos_vec)
            nb_off = no + r * K
            ov0 = noff[pl.ds(nb_off, 16)]
            ov1 = noff[pl.ds(nb_off + 16, 16)]
            v0 = zero16
            v1 = zero16
            for k in range(16):
                v0 = jnp.where(lanes == k,
                               _dotv(cv, nb, r * K + k, ov0[k]), v0)
            for k in range(K - 16):
                v1 = jnp.where(lanes == k,
                               _dotv(cv, nb, r * K + 16 + k, ov1[k]), v1)
            plsc.store_scatter(neg_o, [nb_off + lanes], v0)
            plsc.store_scatter(neg_o, [nb_off + 16 + lanes], v1,
                               mask=lanes < K - 16)
            return pos_vec

        pos_vec = lax.fori_loop(0, CH, row_body, zero16)
        pos_o[pl.ds(ro, CH)] = pos_vec

    fire(0, 0)

    def pair(j2, _):
        j0 = 2 * j2
        drain(j0, 0)
        fire(j0 + 1, 1)
        compute(j0, 0)
        j1 = j0 + 1
        drain(j1, 1)

        @pl.when(j1 < NCHUNK - 1)
        def _():
            fire(j1 + 1, 0)

        compute(j1, 1)
        return 0

    lax.fori_loop(0, NCHUNK // 2, pair, 0)

    pltpu.sync_copy(pos_o, pos_hbm.at[pl.ds(base, RPW)])
    pltpu.sync_copy(neg_o, negsc_hbm.at[pl.ds(nbase, NPW)])


_sc_scores = pl.kernel(
    _sc_body,
    out_type=(
        jax.ShapeDtypeStruct((B,), jnp.float32),
        jax.ShapeDtypeStruct((B * K,), jnp.float32),
    ),
    mesh=plsc.VectorSubcoreMesh(
        core_axis_name="c", subcore_axis_name="s",
        num_cores=NC, num_subcores=NS,
    ),
    compiler_params=pltpu.CompilerParams(needs_layout_passes=False),
    scratch_types=[
        pltpu.VMEM((RPW,), jnp.int32),
        pltpu.VMEM((RPW + 16,), jnp.int32),
        pltpu.VMEM((RPW,), jnp.int32),
        pltpu.VMEM((RPW + 16,), jnp.int32),
        pltpu.VMEM((NPW,), jnp.int32),
        pltpu.VMEM((NPW + 16,), jnp.int32),
        pltpu.VMEM((CH, 2 * D), jnp.float32),
        pltpu.VMEM((CH, 2 * D), jnp.float32),
        pltpu.VMEM((CH, 2 * D), jnp.float32),
        pltpu.VMEM((CH, 2 * D), jnp.float32),
        pltpu.VMEM((NEGC, 2 * D), jnp.float32),
        pltpu.VMEM((NEGC, 2 * D), jnp.float32),
        pltpu.VMEM((RPW,), jnp.float32),
        pltpu.VMEM((NPW,), jnp.float32),
        pltpu.SemaphoreType.DMA,
        pltpu.SemaphoreType.DMA,
    ],
)


def _log_sigmoid(x):
    return jnp.minimum(x, 0.0) - jnp.log1p(jnp.exp(-jnp.abs(x)))


def _loss_body(pos_ref, neg_ref, out_ref):
    ls_pos = jnp.sum(_log_sigmoid(pos_ref[...]))
    ls_neg = jnp.sum(_log_sigmoid(-neg_ref[...]))
    out_ref[0, 0] = -(ls_pos + ls_neg) / B


_loss = pl.pallas_call(
    _loss_body,
    out_shape=jax.ShapeDtypeStruct((1, 1), jnp.float32),
    out_specs=pl.BlockSpec(memory_space=pltpu.SMEM),
)


def _conv_body(a1_ref, a2_ref, b1_ref, b2_ref, oa_ref, ob_ref):
    ident = (lax.broadcasted_iota(jnp.int32, (D, D), 0)
             == lax.broadcasted_iota(jnp.int32, (D, D), 1)).astype(jnp.float32)

    def t(x):
        return lax.dot_general(x, ident, (((0,), (0,)), ((), ())),
                               preferred_element_type=jnp.float32)

    oa_ref[...] = jnp.concatenate([t(a1_ref[...]), t(a2_ref[...])], axis=1)
    ob_ref[...] = jnp.concatenate([t(b1_ref[...]), t(b2_ref[...])], axis=1)


_convert = pl.pallas_call(
    _conv_body,
    grid=(NCONV,),
    in_specs=[
        pl.BlockSpec((D, CONVR), lambda j: (0, 2 * j)),
        pl.BlockSpec((D, CONVR), lambda j: (0, jnp.minimum(2 * j + 1, CLAMPB))),
        pl.BlockSpec((D, CONVR), lambda j: (0, 2 * j)),
        pl.BlockSpec((D, CONVR), lambda j: (0, jnp.minimum(2 * j + 1, CLAMPB))),
    ],
    out_specs=[
        pl.BlockSpec((CONVR, 2 * D), lambda j: (j, 0)),
        pl.BlockSpec((CONVR, 2 * D), lambda j: (j, 0)),
    ],
    out_shape=[
        jax.ShapeDtypeStruct((SROWS, 2 * D), jnp.float32),
        jax.ShapeDtypeStruct((SROWS, 2 * D), jnp.float32),
    ],
    compiler_params=pltpu.CompilerParams(fuse_transposed_lhs_in_matmul=True),
)


@jax.jit
def kernel(centers, positives, negatives, W_in, W_out):
    win_t, wout_t = W_in.T, W_out.T
    s_in, s_out = _convert(win_t, win_t, wout_t, wout_t)
    pos_s, neg_s = _sc_scores(centers, positives, negatives.reshape(-1),
                              s_in, s_out)
    total = _loss(pos_s.reshape(128, B // 128), neg_s.reshape(B * K // 128, 128))
    return total[0, 0]
